# baseline (speedup 1.0000x reference)
import functools
import math

import jax
import jax.numpy as jnp
import numpy as np
from jax import lax
from jax.experimental import pallas as pl
from jax.experimental.pallas import tpu as pltpu
from jax.experimental.pallas import tpu_sc as plsc

_rng = np.random.default_rng(0)
_z = _rng.standard_normal(1_000_000)
_SSP_CONST = float(1.0 / np.sqrt(np.mean((np.logaddexp(0.0, _z) - np.log(2.0)) ** 2)))
_LOG2 = float(np.log(2.0))

_NC = 2
_NS = 16
_NW = _NC * _NS
_CH = 80


_LOG2E = float(1.0 / np.log(2.0))


def _mlp_body(ee_ref, w1_ref, w2_ref, out_ref):
    x = lax.dot_general(ee_ref[...], w1_ref[...], (((0,), (0,)), ((), ())),
                        preferred_element_type=jnp.float32)
    z = x * _LOG2E
    r = jnp.maximum(z, 0.0) + jnp.log2(1.0 + jnp.exp2(-jnp.abs(z)))
    h = (r - 1.0) * (_LOG2 * _SSP_CONST)
    w = jnp.dot(h, w2_ref[...], preferred_element_type=jnp.float32)
    out_ref[...] = w


def _edge_weights(ee_t, W1s, W2s):
    R, E = ee_t.shape
    D = W2s.shape[1]
    BE = 6400
    assert E % BE == 0
    grid = (E // BE,)
    return pl.pallas_call(
        _mlp_body,
        grid=grid,
        in_specs=[
            pl.BlockSpec((R, BE), lambda i: (0, i)),
            pl.BlockSpec(W1s.shape, lambda i: (0, 0)),
            pl.BlockSpec(W2s.shape, lambda i: (0, 0)),
        ],
        out_specs=pl.BlockSpec((BE, D), lambda i: (i, 0)),
        out_shape=jax.ShapeDtypeStruct((E, D), jnp.float32),
    )(ee_t, W1s, W2s)


def _sc_body(node_hbm, w_hbm, ei_hbm, out_hbm,
             src_v, dst_v, rows_v, wv0, wv1, acc_sh,
             gsem0, gsem1, ssem0, ssem1, sisem0, sisem1, dsem0, dsem1):
    N_pad = acc_sh.shape[0]
    D = rows_v.shape[2]
    E = w_hbm.shape[0]
    n_cpt = E // _CH // _NW
    rows_per_tile = N_pad // _NS

    cid = lax.axis_index("c")
    sid = lax.axis_index("s")
    wid = sid * _NC + cid
    gsem = (gsem0, gsem1)
    ssem = (ssem0, ssem1)
    sisem = (sisem0, sisem1)
    dsem = (dsem0, dsem1)
    wv = (wv0, wv1)

    def zrow(r, carry):
        for j in range(D // 16):
            rows_v[0, r, pl.ds(j * 16, 16)] = jnp.zeros((16,), jnp.float32)
        return carry

    lax.fori_loop(0, _CH, zrow, 0)

    row0 = sid * rows_per_tile
    n_zc = rows_per_tile // _CH

    def zacc(b, carry):
        pltpu.async_copy(rows_v.at[0], acc_sh.at[pl.ds(row0 + b * _CH, _CH)],
                         gsem0)
        return carry

    lax.fori_loop(0, n_zc, zacc, 0)

    def zacc_w(b, carry):
        pltpu.make_async_copy(rows_v.at[0], acc_sh.at[pl.ds(row0, _CH)],
                              gsem0).wait()
        return carry

    lax.fori_loop(0, n_zc, zacc_w, 0)

    plsc.subcore_barrier()

    base_e = wid * n_cpt * _CH

    def issue_gather(k, b):
        pltpu.async_copy(node_hbm.at[src_v.at[b]], rows_v.at[b], gsem[b])
        pltpu.async_copy(w_hbm.at[pl.ds(base_e + k * _CH, _CH)], wv[b], gsem[b])

    def wait_gather(b):
        pltpu.make_async_copy(node_hbm.at[src_v.at[b]], rows_v.at[b], gsem[b]).wait()
        pltpu.make_async_copy(w_hbm.at[pl.ds(0, _CH)], wv[b], gsem[b]).wait()

    def wait_scatter(b):
        pltpu.make_async_copy(rows_v.at[b], acc_sh.at[dst_v.at[b]], ssem[b]).wait()

    def issue_src(k, b):
        pltpu.async_copy(ei_hbm.at[pl.ds(E + base_e + k * _CH, _CH)],
                         src_v.at[b], sisem[b])

    def wait_src(b):
        pltpu.make_async_copy(ei_hbm.at[pl.ds(0, _CH)], src_v.at[b], sisem[b]).wait()

    def issue_dst(k, b):
        pltpu.async_copy(ei_hbm.at[pl.ds(base_e + k * _CH, _CH)],
                         dst_v.at[b], dsem[b])

    def wait_dst(b):
        pltpu.make_async_copy(ei_hbm.at[pl.ds(0, _CH)], dst_v.at[b], dsem[b]).wait()

    pltpu.sync_copy(ei_hbm.at[pl.ds(E + base_e, _CH)], src_v.at[0])
    issue_src(1, 1)
    issue_dst(0, 0)
    issue_gather(0, 0)

    def iteration(k, b):
        nb = 1 - b

        @pl.when(k + 1 < n_cpt)
        def _():
            @pl.when(k >= 1)
            def _():
                wait_scatter(nb)

            wait_src(nb)
            issue_gather(k + 1, nb)
            issue_dst(k + 1, nb)

        wait_gather(b)

        @pl.when(k + 2 < n_cpt)
        def _():
            issue_src(k + 2, b)

        def mul_row(r2, cy):
            for rr in range(2):
                r = 2 * r2 + rr
                for j in range(D // 16):
                    sl = pl.ds(j * 16, 16)
                    rows_v[b, r, sl] = rows_v[b, r, sl] * wv[b][r, sl]
            return cy

        lax.fori_loop(0, _CH // 2, mul_row, 0)

        wait_dst(b)
        pltpu.async_copy(rows_v.at[b], acc_sh.at[dst_v.at[b]], ssem[b], add=True)

    def pair_body(k2, carry):
        iteration(2 * k2, 0)
        iteration(2 * k2 + 1, 1)
        return carry

    lax.fori_loop(0, n_cpt // 2, pair_body, 0)
    if n_cpt % 2:
        iteration(jnp.int32(n_cpt - 1), (n_cpt - 1) % 2)
    wait_scatter(0)
    wait_scatter(1)

    plsc.subcore_barrier()

    def wout(b, carry):
        pltpu.async_copy(acc_sh.at[pl.ds(row0 + b * _CH, _CH)],
                         out_hbm.at[cid, pl.ds(row0 + b * _CH, _CH)], gsem0)
        return carry

    lax.fori_loop(0, n_zc, wout, 0)

    def wout_w(b, carry):
        pltpu.make_async_copy(acc_sh.at[pl.ds(row0, _CH)],
                              out_hbm.at[cid, pl.ds(row0, _CH)], gsem0).wait()
        return carry

    lax.fori_loop(0, n_zc, wout_w, 0)


def _sc_scatter(node_features, weight, edge_index):
    N, D = node_features.shape
    E = weight.shape[0]
    n_cpt = E // _CH // _NW
    assert n_cpt * _CH * _NW == E
    ei_flat = edge_index.reshape(2 * E)
    N_pad = ((N + _NS * _CH - 1) // (_NS * _CH)) * (_NS * _CH)
    mesh = plsc.VectorSubcoreMesh(core_axis_name="c", subcore_axis_name="s",
                                  num_cores=_NC, num_subcores=_NS)
    k = functools.partial(
        pl.kernel,
        mesh=mesh,
        out_type=jax.ShapeDtypeStruct((_NC, N_pad, D), jnp.float32),
        scratch_types=[
            pltpu.VMEM((2, _CH), jnp.int32),
            pltpu.VMEM((2, _CH), jnp.int32),
            pltpu.VMEM((2, _CH, D), jnp.float32),
            pltpu.VMEM((_CH, D), jnp.float32),
            pltpu.VMEM((_CH, D), jnp.float32),
            pltpu.VMEM_SHARED((N_pad, D), jnp.float32),
        ] + [pltpu.SemaphoreType.DMA] * 8,
    )(_sc_body)
    return k(node_features, weight, ei_flat)


def _combine_body(p_ref, o_ref):
    o_ref[...] = p_ref[0] + p_ref[1]


def _combine(partial, N):
    D = partial.shape[2]
    BN = 2000
    assert N % BN == 0
    return pl.pallas_call(
        _combine_body,
        grid=(N // BN,),
        in_specs=[pl.BlockSpec((2, BN, D), lambda i: (0, i, 0))],
        out_specs=pl.BlockSpec((BN, D), lambda i: (i, 0)),
        out_shape=jax.ShapeDtypeStruct((N, D), jnp.float32),
    )(partial)


def kernel(node_features, edge_attr, edge_embedding, W1, W2, denominator, edge_index):
    R = W1.shape[0]
    H = W2.shape[0]
    W1s = W1 / jnp.sqrt(jnp.float32(R))
    W2s = W2 / (jnp.sqrt(jnp.float32(H)) * denominator[0])
    weight = _edge_weights(edge_embedding.T, W1s, W2s)
    partial = _sc_scatter(node_features, weight, edge_index)
    return _combine(partial, node_features.shape[0])


# speedup vs baseline: 1.0254x; 1.0254x over previous
import functools
import math

import jax
import jax.numpy as jnp
import numpy as np
from jax import lax
from jax.experimental import pallas as pl
from jax.experimental.pallas import tpu as pltpu
from jax.experimental.pallas import tpu_sc as plsc

_rng = np.random.default_rng(0)
_z = _rng.standard_normal(1_000_000)
_SSP_CONST = float(1.0 / np.sqrt(np.mean((np.logaddexp(0.0, _z) - np.log(2.0)) ** 2)))
_LOG2 = float(np.log(2.0))

_NC = 2
_NS = 16
_NW = _NC * _NS
_CH = 80


_LOG2E = float(1.0 / np.log(2.0))


def _mlp_body(ee_ref, w1_ref, w2_ref, out_ref):
    x = lax.dot_general(ee_ref[...], w1_ref[...], (((0,), (0,)), ((), ())),
                        preferred_element_type=jnp.float32)
    z = x * _LOG2E
    r = jnp.maximum(z, 0.0) + jnp.log2(1.0 + jnp.exp2(-jnp.abs(z)))
    h = (r - 1.0) * (_LOG2 * _SSP_CONST)
    w = jnp.dot(h, w2_ref[...], preferred_element_type=jnp.float32)
    out_ref[...] = w


def _edge_weights(ee_t, W1s, W2s):
    R, E = ee_t.shape
    D = W2s.shape[1]
    BE = 12800
    assert E % BE == 0
    grid = (E // BE,)
    return pl.pallas_call(
        _mlp_body,
        grid=grid,
        in_specs=[
            pl.BlockSpec((R, BE), lambda i: (0, i)),
            pl.BlockSpec(W1s.shape, lambda i: (0, 0)),
            pl.BlockSpec(W2s.shape, lambda i: (0, 0)),
        ],
        out_specs=pl.BlockSpec((BE, D), lambda i: (i, 0)),
        out_shape=jax.ShapeDtypeStruct((E, D), jnp.float32),
    )(ee_t, W1s, W2s)


def _sc_body(node_hbm, w_hbm, ei_hbm, out_hbm,
             src_v, dst_v, rows_v, wv0, wv1, acc_sh,
             gsem0, gsem1, ssem0, ssem1, sisem0, sisem1, dsem0, dsem1):
    N_pad = acc_sh.shape[0]
    D = rows_v.shape[2]
    E = w_hbm.shape[0]
    n_cpt = E // _CH // _NW
    rows_per_tile = N_pad // _NS

    cid = lax.axis_index("c")
    sid = lax.axis_index("s")
    wid = sid * _NC + cid
    gsem = (gsem0, gsem1)
    ssem = (ssem0, ssem1)
    sisem = (sisem0, sisem1)
    dsem = (dsem0, dsem1)
    wv = (wv0, wv1)

    def zrow(r, carry):
        for j in range(D // 16):
            rows_v[0, r, pl.ds(j * 16, 16)] = jnp.zeros((16,), jnp.float32)
        return carry

    lax.fori_loop(0, _CH, zrow, 0)

    row0 = sid * rows_per_tile
    n_zc = rows_per_tile // _CH

    def zacc(b, carry):
        pltpu.async_copy(rows_v.at[0], acc_sh.at[pl.ds(row0 + b * _CH, _CH)],
                         gsem0)
        return carry

    lax.fori_loop(0, n_zc, zacc, 0)

    def zacc_w(b, carry):
        pltpu.make_async_copy(rows_v.at[0], acc_sh.at[pl.ds(row0, _CH)],
                              gsem0).wait()
        return carry

    lax.fori_loop(0, n_zc, zacc_w, 0)

    plsc.subcore_barrier()

    base_e = wid * n_cpt * _CH

    def issue_gather(k, b):
        pltpu.async_copy(node_hbm.at[src_v.at[b]], rows_v.at[b], gsem[b])
        pltpu.async_copy(w_hbm.at[pl.ds(base_e + k * _CH, _CH)], wv[b], gsem[b])

    def wait_gather(b):
        pltpu.make_async_copy(node_hbm.at[src_v.at[b]], rows_v.at[b], gsem[b]).wait()
        pltpu.make_async_copy(w_hbm.at[pl.ds(0, _CH)], wv[b], gsem[b]).wait()

    def wait_scatter(b):
        pltpu.make_async_copy(rows_v.at[b], acc_sh.at[dst_v.at[b]], ssem[b]).wait()

    def issue_src(k, b):
        pltpu.async_copy(ei_hbm.at[pl.ds(E + base_e + k * _CH, _CH)],
                         src_v.at[b], sisem[b])

    def wait_src(b):
        pltpu.make_async_copy(ei_hbm.at[pl.ds(0, _CH)], src_v.at[b], sisem[b]).wait()

    def issue_dst(k, b):
        pltpu.async_copy(ei_hbm.at[pl.ds(base_e + k * _CH, _CH)],
                         dst_v.at[b], dsem[b])

    def wait_dst(b):
        pltpu.make_async_copy(ei_hbm.at[pl.ds(0, _CH)], dst_v.at[b], dsem[b]).wait()

    pltpu.sync_copy(ei_hbm.at[pl.ds(E + base_e, _CH)], src_v.at[0])
    issue_src(1, 1)
    issue_dst(0, 0)
    issue_gather(0, 0)

    def iteration(k, b):
        nb = 1 - b

        @pl.when(k + 1 < n_cpt)
        def _():
            @pl.when(k >= 1)
            def _():
                wait_scatter(nb)

            wait_src(nb)
            issue_gather(k + 1, nb)
            issue_dst(k + 1, nb)

        wait_gather(b)

        @pl.when(k + 2 < n_cpt)
        def _():
            issue_src(k + 2, b)

        def mul_row(r2, cy):
            for rr in range(2):
                r = 2 * r2 + rr
                for j in range(D // 16):
                    sl = pl.ds(j * 16, 16)
                    rows_v[b, r, sl] = rows_v[b, r, sl] * wv[b][r, sl]
            return cy

        lax.fori_loop(0, _CH // 2, mul_row, 0)

        wait_dst(b)
        pltpu.async_copy(rows_v.at[b], acc_sh.at[dst_v.at[b]], ssem[b], add=True)

    def pair_body(k2, carry):
        iteration(2 * k2, 0)
        iteration(2 * k2 + 1, 1)
        return carry

    lax.fori_loop(0, n_cpt // 2, pair_body, 0)
    if n_cpt % 2:
        iteration(jnp.int32(n_cpt - 1), (n_cpt - 1) % 2)
    wait_scatter(0)
    wait_scatter(1)

    plsc.subcore_barrier()

    def wout(b, carry):
        pltpu.async_copy(acc_sh.at[pl.ds(row0 + b * _CH, _CH)],
                         out_hbm.at[cid, pl.ds(row0 + b * _CH, _CH)], gsem0)
        return carry

    lax.fori_loop(0, n_zc, wout, 0)

    def wout_w(b, carry):
        pltpu.make_async_copy(acc_sh.at[pl.ds(row0, _CH)],
                              out_hbm.at[cid, pl.ds(row0, _CH)], gsem0).wait()
        return carry

    lax.fori_loop(0, n_zc, wout_w, 0)


def _sc_scatter(node_features, weight, edge_index):
    N, D = node_features.shape
    E = weight.shape[0]
    n_cpt = E // _CH // _NW
    assert n_cpt * _CH * _NW == E
    ei_flat = edge_index.reshape(2 * E)
    N_pad = ((N + _NS * _CH - 1) // (_NS * _CH)) * (_NS * _CH)
    mesh = plsc.VectorSubcoreMesh(core_axis_name="c", subcore_axis_name="s",
                                  num_cores=_NC, num_subcores=_NS)
    k = functools.partial(
        pl.kernel,
        mesh=mesh,
        out_type=jax.ShapeDtypeStruct((_NC, N_pad, D), jnp.float32),
        scratch_types=[
            pltpu.VMEM((2, _CH), jnp.int32),
            pltpu.VMEM((2, _CH), jnp.int32),
            pltpu.VMEM((2, _CH, D), jnp.float32),
            pltpu.VMEM((_CH, D), jnp.float32),
            pltpu.VMEM((_CH, D), jnp.float32),
            pltpu.VMEM_SHARED((N_pad, D), jnp.float32),
        ] + [pltpu.SemaphoreType.DMA] * 8,
    )(_sc_body)
    return k(node_features, weight, ei_flat)


def _combine_body(p_ref, o_ref):
    o_ref[...] = p_ref[0] + p_ref[1]


def _combine(partial, N):
    D = partial.shape[2]
    BN = 2000
    assert N % BN == 0
    return pl.pallas_call(
        _combine_body,
        grid=(N // BN,),
        in_specs=[pl.BlockSpec((2, BN, D), lambda i: (0, i, 0))],
        out_specs=pl.BlockSpec((BN, D), lambda i: (i, 0)),
        out_shape=jax.ShapeDtypeStruct((N, D), jnp.float32),
    )(partial)


def kernel(node_features, edge_attr, edge_embedding, W1, W2, denominator, edge_index):
    R = W1.shape[0]
    H = W2.shape[0]
    W1s = W1 / jnp.sqrt(jnp.float32(R))
    W2s = W2 / (jnp.sqrt(jnp.float32(H)) * denominator[0])
    weight = _edge_weights(edge_embedding.T, W1s, W2s)
    partial = _sc_scatter(node_features, weight, edge_index)
    return _combine(partial, node_features.shape[0])


# speedup vs baseline: 1.0257x; 1.0003x over previous
import functools

import jax
import jax.numpy as jnp
import numpy as np
from jax import lax
from jax.experimental import pallas as pl
from jax.experimental.pallas import tpu as pltpu
from jax.experimental.pallas import tpu_sc as plsc

_rng = np.random.default_rng(0)
_z = _rng.standard_normal(1_000_000)
_SSP_CONST = float(1.0 / np.sqrt(np.mean((np.logaddexp(0.0, _z) - np.log(2.0)) ** 2)))
_LOG2 = float(np.log(2.0))

_NC = 2
_NS = 16
_NW = _NC * _NS
_CH = 80


_LOG2E = float(1.0 / np.log(2.0))


def _mlp_body(ee_ref, w1_ref, w2_ref, out_ref):
    x = lax.dot_general(ee_ref[...], w1_ref[...], (((0,), (0,)), ((), ())),
                        preferred_element_type=jnp.float32)
    z = x * _LOG2E
    r = jnp.maximum(z, 0.0) + jnp.log2(1.0 + jnp.exp2(-jnp.abs(z)))
    h = (r - 1.0) * (_LOG2 * _SSP_CONST)
    w = jnp.dot(h, w2_ref[...], preferred_element_type=jnp.float32)
    out_ref[...] = w


def _edge_weights(ee_t, W1s, W2s):
    R, E = ee_t.shape
    D = W2s.shape[1]
    BE = 12800
    assert E % BE == 0
    grid = (E // BE,)
    return pl.pallas_call(
        _mlp_body,
        grid=grid,
        in_specs=[
            pl.BlockSpec((R, BE), lambda i: (0, i)),
            pl.BlockSpec(W1s.shape, lambda i: (0, 0)),
            pl.BlockSpec(W2s.shape, lambda i: (0, 0)),
        ],
        out_specs=pl.BlockSpec((BE, D), lambda i: (i, 0)),
        out_shape=jax.ShapeDtypeStruct((E, D), jnp.float32),
    )(ee_t, W1s, W2s)


def _sc_body(node_hbm, w_hbm, ei_hbm, out_hbm,
             src_v, dst_v, rows_v, wv0, wv1, acc_sh,
             gsem0, gsem1, ssem0, ssem1, sisem0, sisem1, dsem0, dsem1):
    N_pad = acc_sh.shape[0]
    D = rows_v.shape[2]
    E = w_hbm.shape[0]
    n_cpt = E // _CH // _NW
    rows_per_tile = N_pad // _NS

    cid = lax.axis_index("c")
    sid = lax.axis_index("s")
    wid = sid * _NC + cid
    gsem = (gsem0, gsem1)
    ssem = (ssem0, ssem1)
    sisem = (sisem0, sisem1)
    dsem = (dsem0, dsem1)
    wv = (wv0, wv1)

    def zrow(r, carry):
        for j in range(D // 16):
            rows_v[0, r, pl.ds(j * 16, 16)] = jnp.zeros((16,), jnp.float32)
        return carry

    lax.fori_loop(0, _CH, zrow, 0)

    row0 = sid * rows_per_tile
    n_zc = rows_per_tile // _CH

    def zacc(b, carry):
        pltpu.async_copy(rows_v.at[0], acc_sh.at[pl.ds(row0 + b * _CH, _CH)],
                         gsem0)
        return carry

    lax.fori_loop(0, n_zc, zacc, 0)

    def zacc_w(b, carry):
        pltpu.make_async_copy(rows_v.at[0], acc_sh.at[pl.ds(row0, _CH)],
                              gsem0).wait()
        return carry

    lax.fori_loop(0, n_zc, zacc_w, 0)

    plsc.subcore_barrier()

    base_e = wid * n_cpt * _CH

    def issue_gather(k, b):
        pltpu.async_copy(node_hbm.at[src_v.at[b]], rows_v.at[b], gsem[b])
        pltpu.async_copy(w_hbm.at[pl.ds(base_e + k * _CH, _CH)], wv[b], gsem[b])

    def wait_gather(b):
        pltpu.make_async_copy(node_hbm.at[src_v.at[b]], rows_v.at[b], gsem[b]).wait()
        pltpu.make_async_copy(w_hbm.at[pl.ds(0, _CH)], wv[b], gsem[b]).wait()

    def wait_scatter(b):
        pltpu.make_async_copy(rows_v.at[b], acc_sh.at[dst_v.at[b]], ssem[b]).wait()

    def issue_src(k, b):
        pltpu.async_copy(ei_hbm.at[pl.ds(E + base_e + k * _CH, _CH)],
                         src_v.at[b], sisem[b])

    def wait_src(b):
        pltpu.make_async_copy(ei_hbm.at[pl.ds(0, _CH)], src_v.at[b], sisem[b]).wait()

    def issue_dst(k, b):
        pltpu.async_copy(ei_hbm.at[pl.ds(base_e + k * _CH, _CH)],
                         dst_v.at[b], dsem[b])

    def wait_dst(b):
        pltpu.make_async_copy(ei_hbm.at[pl.ds(0, _CH)], dst_v.at[b], dsem[b]).wait()

    pltpu.sync_copy(ei_hbm.at[pl.ds(E + base_e, _CH)], src_v.at[0])
    issue_src(1, 1)
    issue_dst(0, 0)
    issue_gather(0, 0)

    def iteration(k, b):
        nb = 1 - b

        @pl.when(k + 1 < n_cpt)
        def _():
            @pl.when(k >= 1)
            def _():
                wait_scatter(nb)

            wait_src(nb)
            issue_gather(k + 1, nb)
            issue_dst(k + 1, nb)

        wait_gather(b)

        @pl.when(k + 2 < n_cpt)
        def _():
            issue_src(k + 2, b)

        def mul_row(r2, cy):
            for rr in range(2):
                r = 2 * r2 + rr
                for j in range(D // 16):
                    sl = pl.ds(j * 16, 16)
                    rows_v[b, r, sl] = rows_v[b, r, sl] * wv[b][r, sl]
            return cy

        lax.fori_loop(0, _CH // 2, mul_row, 0)

        wait_dst(b)
        pltpu.async_copy(rows_v.at[b], acc_sh.at[dst_v.at[b]], ssem[b], add=True)

    def pair_body(k2, carry):
        iteration(2 * k2, 0)
        iteration(2 * k2 + 1, 1)
        return carry

    lax.fori_loop(0, n_cpt // 2, pair_body, 0)
    if n_cpt % 2:
        iteration(jnp.int32(n_cpt - 1), (n_cpt - 1) % 2)
    wait_scatter(0)
    wait_scatter(1)

    plsc.subcore_barrier()

    def wout(b, carry):
        pltpu.async_copy(acc_sh.at[pl.ds(row0 + b * _CH, _CH)],
                         out_hbm.at[cid, pl.ds(row0 + b * _CH, _CH)], gsem0)
        return carry

    lax.fori_loop(0, n_zc, wout, 0)

    def wout_w(b, carry):
        pltpu.make_async_copy(acc_sh.at[pl.ds(row0, _CH)],
                              out_hbm.at[cid, pl.ds(row0, _CH)], gsem0).wait()
        return carry

    lax.fori_loop(0, n_zc, wout_w, 0)


def _sc_scatter(node_features, weight, edge_index):
    N, D = node_features.shape
    E = weight.shape[0]
    n_cpt = E // _CH // _NW
    assert n_cpt * _CH * _NW == E
    ei_flat = edge_index.reshape(2 * E)
    N_pad = ((N + _NS * _CH - 1) // (_NS * _CH)) * (_NS * _CH)
    mesh = plsc.VectorSubcoreMesh(core_axis_name="c", subcore_axis_name="s",
                                  num_cores=_NC, num_subcores=_NS)
    k = functools.partial(
        pl.kernel,
        mesh=mesh,
        out_type=jax.ShapeDtypeStruct((_NC, N_pad, D), jnp.float32),
        scratch_types=[
            pltpu.VMEM((2, _CH), jnp.int32),
            pltpu.VMEM((2, _CH), jnp.int32),
            pltpu.VMEM((2, _CH, D), jnp.float32),
            pltpu.VMEM((_CH, D), jnp.float32),
            pltpu.VMEM((_CH, D), jnp.float32),
            pltpu.VMEM_SHARED((N_pad, D), jnp.float32),
        ] + [pltpu.SemaphoreType.DMA] * 8,
    )(_sc_body)
    return k(node_features, weight, ei_flat)


def _combine_body(p_ref, o_ref):
    o_ref[...] = p_ref[0] + p_ref[1]


def _combine(partial, N):
    D = partial.shape[2]
    BN = 2000
    assert N % BN == 0
    return pl.pallas_call(
        _combine_body,
        grid=(N // BN,),
        in_specs=[pl.BlockSpec((2, BN, D), lambda i: (0, i, 0))],
        out_specs=pl.BlockSpec((BN, D), lambda i: (i, 0)),
        out_shape=jax.ShapeDtypeStruct((N, D), jnp.float32),
    )(partial)


def kernel(node_features, edge_attr, edge_embedding, W1, W2, denominator, edge_index):
    R = W1.shape[0]
    H = W2.shape[0]
    W1s = W1 / jnp.sqrt(jnp.float32(R))
    W2s = W2 / (jnp.sqrt(jnp.float32(H)) * denominator[0])
    weight = _edge_weights(edge_embedding.T, W1s, W2s)
    partial = _sc_scatter(node_features, weight, edge_index)
    return _combine(partial, node_features.shape[0])
